# baseline (speedup 1.0000x reference)
import functools

import jax
import jax.numpy as jnp
from jax.experimental import pallas as pl


def _vq_block(x_ref, cb_ref, cbta_ref, q_ref, idx_ref, loss_ref, *, blk_n):
    b = pl.program_id(0)
    j = pl.program_id(1)

    xb = x_ref[...]
    cb = cb_ref[...]

    xb16 = xb.astype(jnp.bfloat16)
    cb16 = cb.astype(jnp.bfloat16)
    dots = jax.lax.dot_general(
        cb16, xb16, (((1,), (0,)), ((), ())),
        preferred_element_type=jnp.float32)

    c_sqh = 0.5 * jnp.sum(cb * cb, axis=1)
    t = dots - c_sqh[:, None]

    maxval = jnp.max(t, axis=0)
    onehot = (t == maxval[None, :]).astype(jnp.bfloat16)

    res = jax.lax.dot_general(cbta_ref[...], onehot, (((1,), (0,)), ((), ())),
                              preferred_element_type=jnp.float32)
    q = res[0:32, :] + res[40:72, :]
    q_ref[...] = q
    idx_ref[...] = (2.0 * res[32:33, :] + res[33:34, :]).astype(jnp.int32)

    part = jnp.sum((q - xb) ** 2).reshape(1, 1)

    @pl.when(jnp.logical_and(b == 0, j == 0))
    def _():
        loss_ref[...] = jnp.zeros_like(loss_ref)

    loss_ref[...] += part


@jax.jit
def kernel(x, codebook):
    B, C, H, W = x.shape
    K = codebook.shape[0]
    N = H * W
    blk_n = 2048

    xr = x.reshape(B, C, N)
    k_idx = jnp.arange(K, dtype=jnp.float32)
    cbt = codebook.T
    cbt_hi32 = jax.lax.bitcast_convert_type(
        jax.lax.bitcast_convert_type(cbt, jnp.uint32) & jnp.uint32(0xFFFF0000),
        jnp.float32)
    cbt_hi = cbt_hi32.astype(jnp.bfloat16)
    cbt_lo = (cbt - cbt_hi32).astype(jnp.bfloat16)
    cbta = jnp.concatenate(
        [cbt_hi,
         jnp.floor(k_idx / 2.0)[None, :].astype(jnp.bfloat16),
         (k_idx % 2.0)[None, :].astype(jnp.bfloat16),
         jnp.zeros((6, K), jnp.bfloat16),
         cbt_lo],
        axis=0)

    grid = (B, N // blk_n)
    q, idx, loss_sum = pl.pallas_call(
        functools.partial(_vq_block, blk_n=blk_n),
        grid=grid,
        in_specs=[
            pl.BlockSpec((None, C, blk_n), lambda b, j: (b, 0, j)),
            pl.BlockSpec((K, C), lambda b, j: (0, 0)),
            pl.BlockSpec((2 * C + 8, K), lambda b, j: (0, 0)),
        ],
        out_specs=[
            pl.BlockSpec((None, C, blk_n), lambda b, j: (b, 0, j)),
            pl.BlockSpec((None, 1, blk_n), lambda b, j: (b, 0, j)),
            pl.BlockSpec((1, 1), lambda b, j: (0, 0)),
        ],
        out_shape=[
            jax.ShapeDtypeStruct((B, C, N), jnp.float32),
            jax.ShapeDtypeStruct((B, 1, N), jnp.int32),
            jax.ShapeDtypeStruct((1, 1), jnp.float32),
        ],
    )(xr, codebook, cbta)

    quantized = q.reshape(B, C, H, W)
    indices = idx.reshape(B, H, W)
    commit_loss = (loss_sum[0, 0] / (B * N * C)).reshape(())
    return quantized, indices, commit_loss
